# baseline (speedup 1.0000x reference)
import functools

import jax
import jax.numpy as jnp
from jax import lax
from jax.experimental import pallas as pl
from jax.experimental.pallas import tpu as pltpu
from jax.experimental.pallas import tpu_sc as plsc

N = 10000
E = 320000
D_IN = 1552
EMB = 128
OUT = 64
Q = 32

NC = 2
NS = 16
CHUNK = 80
SB = 16
NSB = 8
K_PER_WORKER = SB * NSB
E_PER_WORKER = K_PER_WORKER * CHUNK
E_PAD = E_PER_WORKER * NC * NS
ROWS_PER_TILE = 624
TAIL_ROWS = N - NS * ROWS_PER_TILE


def _enc_body(u_ref, wbd_ref, b_ref, w1_ref, out_ref):
    y = jnp.dot(u_ref[...], wbd_ref[...], preferred_element_type=jnp.float32)
    y = y + b_ref[...]
    y = jnp.where(y >= 0, y, 0.01 * y)
    out_ref[...] = jnp.dot(y, w1_ref[...], preferred_element_type=jnp.float32)


def _mid_body(p_ref, w2_ref, out_ref):
    z = p_ref[0] + p_ref[1]
    out_ref[...] = jnp.dot(z, w2_ref[...], preferred_element_type=jnp.float32)


def _zmean_body(q_ref, out_ref):
    out_ref[...] = q_ref[0, :, :OUT] + q_ref[1, :, :OUT]


def _decoder_body(gamma_ref, zi_ref, zj_ref, rec_ref, cd_ref):
    zi = zi_ref[...]
    zj = zj_ref[...]
    rec = lax.dot_general(zi, zj, (((1,), (1,)), ((), ())),
                          preferred_element_type=jnp.float32)
    rec_ref[...] = rec
    sqi = jnp.sum(zi * zi, axis=1, keepdims=True)
    ones = jnp.ones((1, OUT), dtype=jnp.float32)
    sqj = lax.dot_general(ones, zj * zj, (((1,), (1,)), ((), ())),
                          preferred_element_type=jnp.float32)
    d2 = jnp.maximum(sqi + sqj - 2.0 * rec, 0.0)
    cd_ref[...] = jnp.exp(-gamma_ref[0, 0] * d2)


def _make_spmm(d, scale_groups=None):
    n_groups = (d // 16) if scale_groups is None else scale_groups
    mesh = plsc.VectorSubcoreMesh(core_axis_name="c", subcore_axis_name="s",
                                  num_cores=NC, num_subcores=NS)
    sb_edges = SB * CHUNK

    @functools.partial(
        pl.kernel,
        out_type=jax.ShapeDtypeStruct((NC, N, d), jnp.float32),
        mesh=mesh,
        scratch_types=[
            pltpu.VMEM_SHARED((N, d), jnp.float32),
            [pltpu.VMEM((sb_edges,), jnp.int32) for _ in range(2)],
            [pltpu.VMEM((SB, CHUNK), jnp.int32) for _ in range(2)],
            [pltpu.VMEM((sb_edges,), jnp.float32) for _ in range(2)],
            pltpu.VMEM((CHUNK, d), jnp.float32),
            pltpu.SemaphoreType.DMA,
            [pltpu.SemaphoreType.DMA for _ in range(2)],
        ],
    )
    def spmm(x_hbm, src_hbm, dst_hbm, w_hbm, zero_hbm, out_hbm,
             acc_sh, src_v, dst_v, w_v, rows, gsem, msem):
        cid = lax.axis_index("c")
        sid = lax.axis_index("s")
        wid = cid * NS + sid

        pltpu.sync_copy(zero_hbm.at[pl.ds(sid * ROWS_PER_TILE, ROWS_PER_TILE)],
                        acc_sh.at[pl.ds(sid * ROWS_PER_TILE, ROWS_PER_TILE)])

        @pl.when(sid == 0)
        def _():
            pltpu.sync_copy(zero_hbm.at[pl.ds(NS * ROWS_PER_TILE, TAIL_ROWS)],
                            acc_sh.at[pl.ds(NS * ROWS_PER_TILE, TAIL_ROWS)])

        def meta_copies(sb, par):
            base = wid * E_PER_WORKER + sb * sb_edges
            return (
                pltpu.make_async_copy(src_hbm.at[pl.ds(base, sb_edges)],
                                      src_v[par], msem[par]),
                pltpu.make_async_copy(dst_hbm.at[wid, pl.ds(sb * SB, SB)],
                                      dst_v[par], msem[par]),
                pltpu.make_async_copy(w_hbm.at[pl.ds(base, sb_edges)],
                                      w_v[par], msem[par]),
            )

        def run_superblock(sb, par):
            for c in meta_copies(sb, par):
                c.wait()

            @pl.when(sb + 1 < NSB)
            def _():
                for c in meta_copies(sb + 1, 1 - par):
                    c.start()

            def chunk_body(local, c1):
                pltpu.async_copy(
                    x_hbm.at[src_v[par].at[pl.ds(local * CHUNK, CHUNK)]],
                    rows, gsem).wait()

                def group_body(grp, c2):
                    wv = w_v[par][pl.ds(local * CHUNK + grp * 16, 16)]
                    for l in range(16):
                        e = grp * 16 + l
                        for g in range(n_groups):
                            rows[e, pl.ds(g * 16, 16)] = (
                                rows[e, pl.ds(g * 16, 16)] * wv[l])
                    return c2

                lax.fori_loop(0, CHUNK // 16, group_body, 0)
                pltpu.sync_copy(rows, acc_sh.at[dst_v[par].at[local]], add=True)
                return c1

            lax.fori_loop(0, SB, chunk_body, 0)

        for c in meta_copies(0, 0):
            c.start()
        plsc.subcore_barrier()

        def sb_pair(p, carry):
            run_superblock(2 * p, 0)
            run_superblock(2 * p + 1, 1)
            return carry

        lax.fori_loop(0, NSB // 2, sb_pair, 0)
        plsc.subcore_barrier()
        pltpu.sync_copy(acc_sh.at[pl.ds(sid * ROWS_PER_TILE, ROWS_PER_TILE)],
                        out_hbm.at[cid, pl.ds(sid * ROWS_PER_TILE, ROWS_PER_TILE)])

        @pl.when(sid == 0)
        def _():
            pltpu.sync_copy(acc_sh.at[pl.ds(NS * ROWS_PER_TILE, TAIL_ROWS)],
                            out_hbm.at[cid, pl.ds(NS * ROWS_PER_TILE, TAIL_ROWS)])

    return spmm


_spmm_full = _make_spmm(EMB)
_spmm_half = _make_spmm(EMB, scale_groups=OUT // 16)


def kernel(user_feature, edge_index, edge_weight, gamma,
           W_num, b_num, W_cat, b_cat, W_des, b_des, W_tweet, b_tweet,
           W1, W2):
    f32 = jnp.float32
    wbd = jnp.zeros((D_IN, EMB), dtype=f32)
    wbd = wbd.at[0:5, 0:Q].set(W_num.T)
    wbd = wbd.at[5:16, Q:2 * Q].set(W_cat.T)
    wbd = wbd.at[16:784, 2 * Q:3 * Q].set(W_des.T)
    wbd = wbd.at[784:1552, 3 * Q:].set(W_tweet.T)
    bias = jnp.concatenate([b_num, b_cat, b_des, b_tweet]).reshape(1, EMB)

    pad = E_PAD - E
    pad_rows = (jnp.arange(pad, dtype=jnp.int32)) % N
    src = jnp.pad(edge_index[1].astype(jnp.int32), (0, pad))
    dst = jnp.concatenate(
        [edge_index[0].astype(jnp.int32), pad_rows]).reshape(
        NC * NS, K_PER_WORKER, CHUNK)
    ew = jnp.pad(edge_weight.astype(f32), (0, pad))
    zeros128 = jnp.zeros((N, EMB), dtype=f32)
    w2p = jnp.zeros((EMB, EMB), dtype=f32).at[:, :OUT].set(W2)

    tm = 1000
    h = pl.pallas_call(
        _enc_body,
        grid=(N // tm,),
        in_specs=[
            pl.BlockSpec((tm, D_IN), lambda i: (i, 0)),
            pl.BlockSpec((D_IN, EMB), lambda i: (0, 0)),
            pl.BlockSpec((1, EMB), lambda i: (0, 0)),
            pl.BlockSpec((EMB, EMB), lambda i: (0, 0)),
        ],
        out_specs=pl.BlockSpec((tm, EMB), lambda i: (i, 0)),
        out_shape=jax.ShapeDtypeStruct((N, EMB), f32),
    )(user_feature, wbd, bias, W1)

    p = _spmm_full(h, src, dst, ew, zeros128)

    m = pl.pallas_call(
        _mid_body,
        grid=(N // tm,),
        in_specs=[
            pl.BlockSpec((NC, tm, EMB), lambda i: (0, i, 0)),
            pl.BlockSpec((EMB, EMB), lambda i: (0, 0)),
        ],
        out_specs=pl.BlockSpec((tm, EMB), lambda i: (i, 0)),
        out_shape=jax.ShapeDtypeStruct((N, EMB), f32),
    )(p, w2p)

    q = _spmm_half(m, src, dst, ew, zeros128)

    z_mean = pl.pallas_call(
        _zmean_body,
        grid=(N // tm,),
        in_specs=[pl.BlockSpec((NC, tm, EMB), lambda i: (0, i, 0))],
        out_specs=pl.BlockSpec((tm, OUT), lambda i: (i, 0)),
        out_shape=jax.ShapeDtypeStruct((N, OUT), f32),
    )(q)

    gamma2d = jnp.asarray(gamma, dtype=f32).reshape(1, 1)
    tdec = 200
    reconstructions, clusters_distance = pl.pallas_call(
        _decoder_body,
        grid=(N // tdec,),
        in_specs=[
            pl.BlockSpec(memory_space=pltpu.SMEM),
            pl.BlockSpec((tdec, OUT), lambda i: (i, 0)),
            pl.BlockSpec((N, OUT), lambda i: (0, 0)),
        ],
        out_specs=[
            pl.BlockSpec((tdec, N), lambda i: (i, 0)),
            pl.BlockSpec((tdec, N), lambda i: (i, 0)),
        ],
        out_shape=[
            jax.ShapeDtypeStruct((N, N), f32),
            jax.ShapeDtypeStruct((N, N), f32),
        ],
    )(gamma2d, z_mean, z_mean)

    return reconstructions, clusters_distance, z_mean


# speedup vs baseline: 1.9924x; 1.9924x over previous
import functools

import jax
import jax.numpy as jnp
from jax import lax
from jax.experimental import pallas as pl
from jax.experimental.pallas import tpu as pltpu
from jax.experimental.pallas import tpu_sc as plsc

N = 10000
E = 320000
D_IN = 1552
EMB = 128
OUT = 64
Q = 32

NC = 2
NS = 16
CHUNK = 80
SB = 16
NSB = 8
K_PER_WORKER = SB * NSB
E_PER_WORKER = K_PER_WORKER * CHUNK
E_PAD = E_PER_WORKER * NC * NS
ROWS_PER_TILE = 624
TAIL_ROWS = N - NS * ROWS_PER_TILE


def _enc_body(u_ref, wbd_ref, b_ref, w1_ref, out_ref):
    y = jnp.dot(u_ref[...], wbd_ref[...], preferred_element_type=jnp.float32)
    y = y + b_ref[...]
    y = jnp.where(y >= 0, y, 0.01 * y)
    out_ref[...] = jnp.dot(y, w1_ref[...], preferred_element_type=jnp.float32)


def _mid_body(p_ref, w2_ref, out_ref):
    z = p_ref[0] + p_ref[1]
    out_ref[...] = jnp.dot(z, w2_ref[...], preferred_element_type=jnp.float32)


def _zmean_body(q_ref, out_ref):
    out_ref[...] = q_ref[0, :, :OUT] + q_ref[1, :, :OUT]


def _decoder_body(gamma_ref, zi_ref, zj_ref, rec_ref, cd_ref):
    zi = zi_ref[...]
    zj = zj_ref[...]
    rec = lax.dot_general(zi, zj, (((1,), (1,)), ((), ())),
                          preferred_element_type=jnp.float32)
    rec_ref[...] = rec
    sqi = jnp.sum(zi * zi, axis=1, keepdims=True)
    ones = jnp.ones((1, OUT), dtype=jnp.float32)
    sqj = lax.dot_general(ones, zj * zj, (((1,), (1,)), ((), ())),
                          preferred_element_type=jnp.float32)
    d2 = jnp.maximum(sqi + sqj - 2.0 * rec, 0.0)
    cd_ref[...] = jnp.exp(-gamma_ref[0, 0] * d2)


def _make_spmm(d, scale_groups=None):
    n_groups = (d // 16) if scale_groups is None else scale_groups
    mesh = plsc.VectorSubcoreMesh(core_axis_name="c", subcore_axis_name="s",
                                  num_cores=NC, num_subcores=NS)
    sb_edges = SB * CHUNK

    @functools.partial(
        pl.kernel,
        out_type=jax.ShapeDtypeStruct((NC, N, d), jnp.float32),
        mesh=mesh,
        scratch_types=[
            pltpu.VMEM_SHARED((N, d), jnp.float32),
            [pltpu.VMEM((sb_edges,), jnp.int32) for _ in range(2)],
            [pltpu.VMEM((SB, CHUNK), jnp.int32) for _ in range(2)],
            [pltpu.VMEM((sb_edges,), jnp.float32) for _ in range(2)],
            pltpu.VMEM((CHUNK, d), jnp.float32),
            pltpu.SemaphoreType.DMA,
            [pltpu.SemaphoreType.DMA for _ in range(2)],
        ],
    )
    def spmm(x_hbm, src_hbm, dst_hbm, w_hbm, zero_hbm, out_hbm,
             acc_sh, src_v, dst_v, w_v, rows, gsem, msem):
        cid = lax.axis_index("c")
        sid = lax.axis_index("s")
        wid = cid * NS + sid

        pltpu.sync_copy(zero_hbm.at[pl.ds(sid * ROWS_PER_TILE, ROWS_PER_TILE)],
                        acc_sh.at[pl.ds(sid * ROWS_PER_TILE, ROWS_PER_TILE)])

        @pl.when(sid == 0)
        def _():
            pltpu.sync_copy(zero_hbm.at[pl.ds(NS * ROWS_PER_TILE, TAIL_ROWS)],
                            acc_sh.at[pl.ds(NS * ROWS_PER_TILE, TAIL_ROWS)])

        def meta_copies(sb, par):
            base = wid * E_PER_WORKER + sb * sb_edges
            return (
                pltpu.make_async_copy(src_hbm.at[pl.ds(base, sb_edges)],
                                      src_v[par], msem[par]),
                pltpu.make_async_copy(dst_hbm.at[wid, pl.ds(sb * SB, SB)],
                                      dst_v[par], msem[par]),
                pltpu.make_async_copy(w_hbm.at[pl.ds(base, sb_edges)],
                                      w_v[par], msem[par]),
            )

        def run_superblock(sb, par):
            for c in meta_copies(sb, par):
                c.wait()

            @pl.when(sb + 1 < NSB)
            def _():
                for c in meta_copies(sb + 1, 1 - par):
                    c.start()

            def chunk_body(local, c1):
                pltpu.async_copy(
                    x_hbm.at[src_v[par].at[pl.ds(local * CHUNK, CHUNK)]],
                    rows, gsem).wait()

                def group_body(grp, c2):
                    wv = w_v[par][pl.ds(local * CHUNK + grp * 16, 16)]
                    for l in range(16):
                        e = grp * 16 + l
                        for g in range(n_groups):
                            rows[e, pl.ds(g * 16, 16)] = (
                                rows[e, pl.ds(g * 16, 16)] * wv[l])
                    return c2

                lax.fori_loop(0, CHUNK // 16, group_body, 0)
                pltpu.sync_copy(rows, acc_sh.at[dst_v[par].at[local]], add=True)
                return c1

            lax.fori_loop(0, SB, chunk_body, 0)

        for c in meta_copies(0, 0):
            c.start()
        plsc.subcore_barrier()

        def sb_pair(p, carry):
            run_superblock(2 * p, 0)
            run_superblock(2 * p + 1, 1)
            return carry

        lax.fori_loop(0, NSB // 2, sb_pair, 0)
        plsc.subcore_barrier()
        pltpu.sync_copy(acc_sh.at[pl.ds(sid * ROWS_PER_TILE, ROWS_PER_TILE)],
                        out_hbm.at[cid, pl.ds(sid * ROWS_PER_TILE, ROWS_PER_TILE)])

        @pl.when(sid == 0)
        def _():
            pltpu.sync_copy(acc_sh.at[pl.ds(NS * ROWS_PER_TILE, TAIL_ROWS)],
                            out_hbm.at[cid, pl.ds(NS * ROWS_PER_TILE, TAIL_ROWS)])

    return spmm


_spmm_full = _make_spmm(EMB)
_spmm_half = _make_spmm(EMB, scale_groups=OUT // 16)


def kernel(user_feature, edge_index, edge_weight, gamma,
           W_num, b_num, W_cat, b_cat, W_des, b_des, W_tweet, b_tweet,
           W1, W2):
    f32 = jnp.float32
    wbd = jnp.zeros((D_IN, EMB), dtype=f32)
    wbd = wbd.at[0:5, 0:Q].set(W_num.T)
    wbd = wbd.at[5:16, Q:2 * Q].set(W_cat.T)
    wbd = wbd.at[16:784, 2 * Q:3 * Q].set(W_des.T)
    wbd = wbd.at[784:1552, 3 * Q:].set(W_tweet.T)
    bias = jnp.concatenate([b_num, b_cat, b_des, b_tweet]).reshape(1, EMB)

    pad = E_PAD - E
    pad_rows = (jnp.arange(pad, dtype=jnp.int32)) % N
    src = jnp.concatenate([edge_index[1].astype(jnp.int32), pad_rows])
    dst = jnp.concatenate(
        [edge_index[0].astype(jnp.int32), pad_rows]).reshape(
        NC * NS, K_PER_WORKER, CHUNK)
    ew = jnp.pad(edge_weight.astype(f32), (0, pad))
    zeros128 = jnp.zeros((N, EMB), dtype=f32)
    w2p = jnp.zeros((EMB, EMB), dtype=f32).at[:, :OUT].set(W2)

    tm = 1000
    h = pl.pallas_call(
        _enc_body,
        grid=(N // tm,),
        in_specs=[
            pl.BlockSpec((tm, D_IN), lambda i: (i, 0)),
            pl.BlockSpec((D_IN, EMB), lambda i: (0, 0)),
            pl.BlockSpec((1, EMB), lambda i: (0, 0)),
            pl.BlockSpec((EMB, EMB), lambda i: (0, 0)),
        ],
        out_specs=pl.BlockSpec((tm, EMB), lambda i: (i, 0)),
        out_shape=jax.ShapeDtypeStruct((N, EMB), f32),
    )(user_feature, wbd, bias, W1)

    p = _spmm_full(h, src, dst, ew, zeros128)

    m = pl.pallas_call(
        _mid_body,
        grid=(N // tm,),
        in_specs=[
            pl.BlockSpec((NC, tm, EMB), lambda i: (0, i, 0)),
            pl.BlockSpec((EMB, EMB), lambda i: (0, 0)),
        ],
        out_specs=pl.BlockSpec((tm, EMB), lambda i: (i, 0)),
        out_shape=jax.ShapeDtypeStruct((N, EMB), f32),
    )(p, w2p)

    q = _spmm_half(m, src, dst, ew, zeros128)

    z_mean = pl.pallas_call(
        _zmean_body,
        grid=(N // tm,),
        in_specs=[pl.BlockSpec((NC, tm, EMB), lambda i: (0, i, 0))],
        out_specs=pl.BlockSpec((tm, OUT), lambda i: (i, 0)),
        out_shape=jax.ShapeDtypeStruct((N, OUT), f32),
    )(q)

    gamma2d = jnp.asarray(gamma, dtype=f32).reshape(1, 1)
    tdec = 200
    reconstructions, clusters_distance = pl.pallas_call(
        _decoder_body,
        grid=(N // tdec,),
        in_specs=[
            pl.BlockSpec(memory_space=pltpu.SMEM),
            pl.BlockSpec((tdec, OUT), lambda i: (i, 0)),
            pl.BlockSpec((N, OUT), lambda i: (0, 0)),
        ],
        out_specs=[
            pl.BlockSpec((tdec, N), lambda i: (i, 0)),
            pl.BlockSpec((tdec, N), lambda i: (i, 0)),
        ],
        out_shape=[
            jax.ShapeDtypeStruct((N, N), f32),
            jax.ShapeDtypeStruct((N, N), f32),
        ],
    )(gamma2d, z_mean, z_mean)

    return reconstructions, clusters_distance, z_mean


# speedup vs baseline: 2.5649x; 1.2874x over previous
import functools

import jax
import jax.numpy as jnp
from jax import lax
from jax.experimental import pallas as pl
from jax.experimental.pallas import tpu as pltpu
from jax.experimental.pallas import tpu_sc as plsc

N = 10000
E = 320000
D_IN = 1552
EMB = 128
OUT = 64
Q = 32

NC = 2
NS = 16
CHUNK = 80
SB = 16
NSB = 8
K_PER_WORKER = SB * NSB
E_PER_WORKER = K_PER_WORKER * CHUNK
E_PAD = E_PER_WORKER * NC * NS
ROWS_PER_TILE = 624
TAIL_ROWS = N - NS * ROWS_PER_TILE


def _enc_body(u_ref, wbd_ref, b_ref, w1_ref, out_ref):
    y = jnp.dot(u_ref[...], wbd_ref[...], preferred_element_type=jnp.float32)
    y = y + b_ref[...]
    y = jnp.where(y >= 0, y, 0.01 * y)
    out_ref[...] = jnp.dot(y, w1_ref[...], preferred_element_type=jnp.float32)


def _mid_body(p_ref, w2_ref, out_ref):
    z = p_ref[0] + p_ref[1]
    out_ref[...] = jnp.dot(z, w2_ref[...], preferred_element_type=jnp.float32)


def _zmean_body(q_ref, out_ref):
    out_ref[...] = q_ref[0, :, :OUT] + q_ref[1, :, :OUT]


def _decoder_body(gamma_ref, zi_ref, zj_ref, rec_ref, cd_ref):
    zi = zi_ref[...]
    zj = zj_ref[...]
    rec = lax.dot_general(zi, zj, (((1,), (1,)), ((), ())),
                          preferred_element_type=jnp.float32)
    rec_ref[...] = rec
    sqi = jnp.sum(zi * zi, axis=1, keepdims=True)
    ones = jnp.ones((1, OUT), dtype=jnp.float32)
    sqj = lax.dot_general(ones, zj * zj, (((1,), (1,)), ((), ())),
                          preferred_element_type=jnp.float32)
    d2 = jnp.maximum(sqi + sqj - 2.0 * rec, 0.0)
    cd_ref[...] = jnp.exp(-gamma_ref[0, 0] * d2)


def _make_spmm(d, scale_groups=None):
    n_groups = (d // 16) if scale_groups is None else scale_groups
    mesh = plsc.VectorSubcoreMesh(core_axis_name="c", subcore_axis_name="s",
                                  num_cores=NC, num_subcores=NS)
    sb_edges = SB * CHUNK

    @functools.partial(
        pl.kernel,
        out_type=jax.ShapeDtypeStruct((NC, N, d), jnp.float32),
        mesh=mesh,
        scratch_types=[
            pltpu.VMEM_SHARED((N, d), jnp.float32),
            [pltpu.VMEM((sb_edges,), jnp.int32) for _ in range(2)],
            [pltpu.VMEM((SB, CHUNK), jnp.int32) for _ in range(2)],
            [pltpu.VMEM((sb_edges,), jnp.float32) for _ in range(2)],
            [pltpu.VMEM((CHUNK, d), jnp.float32) for _ in range(2)],
            [pltpu.SemaphoreType.DMA for _ in range(2)],
            [pltpu.SemaphoreType.DMA for _ in range(2)],
        ],
    )
    def spmm(x_hbm, src_hbm, dst_hbm, w_hbm, zero_hbm, out_hbm,
             acc_sh, src_v, dst_v, w_v, rows, gsem, msem):
        cid = lax.axis_index("c")
        sid = lax.axis_index("s")
        wid = cid * NS + sid

        pltpu.sync_copy(zero_hbm.at[pl.ds(sid * ROWS_PER_TILE, ROWS_PER_TILE)],
                        acc_sh.at[pl.ds(sid * ROWS_PER_TILE, ROWS_PER_TILE)])

        @pl.when(sid == 0)
        def _():
            pltpu.sync_copy(zero_hbm.at[pl.ds(NS * ROWS_PER_TILE, TAIL_ROWS)],
                            acc_sh.at[pl.ds(NS * ROWS_PER_TILE, TAIL_ROWS)])

        def meta_copies(sb, par):
            base = wid * E_PER_WORKER + sb * sb_edges
            return (
                pltpu.make_async_copy(src_hbm.at[pl.ds(base, sb_edges)],
                                      src_v[par], msem[par]),
                pltpu.make_async_copy(dst_hbm.at[wid, pl.ds(sb * SB, SB)],
                                      dst_v[par], msem[par]),
                pltpu.make_async_copy(w_hbm.at[pl.ds(base, sb_edges)],
                                      w_v[par], msem[par]),
            )

        def run_superblock(sb, par):
            for c in meta_copies(sb, par):
                c.wait()

            @pl.when(sb + 1 < NSB)
            def _():
                for c in meta_copies(sb + 1, 1 - par):
                    c.start()

            def fire_gather(local, b):
                pltpu.async_copy(
                    x_hbm.at[src_v[par].at[pl.ds(local * CHUNK, CHUNK)]],
                    rows[b], gsem[b])

            def wait_gather(local, b):
                pltpu.make_async_copy(
                    x_hbm.at[src_v[par].at[pl.ds(local * CHUNK, CHUNK)]],
                    rows[b], gsem[b]).wait()

            def process(local, b):
                wait_gather(local, b)

                def group_body(grp, c2):
                    wv = w_v[par][pl.ds(local * CHUNK + grp * 16, 16)]
                    for l in range(16):
                        e = grp * 16 + l
                        for g in range(n_groups):
                            rows[b][e, pl.ds(g * 16, 16)] = (
                                rows[b][e, pl.ds(g * 16, 16)] * wv[l])
                    return c2

                lax.fori_loop(0, CHUNK // 16, group_body, 0)
                pltpu.sync_copy(rows[b], acc_sh.at[dst_v[par].at[local]],
                                add=True)

            fire_gather(0, 0)

            def round_body(r, c1):
                for b in range(2):
                    local = r * 2 + b

                    @pl.when(local + 1 < SB)
                    def _():
                        fire_gather(local + 1, 1 - b)

                    process(local, b)
                return c1

            lax.fori_loop(0, SB // 2, round_body, 0)

        for c in meta_copies(0, 0):
            c.start()
        plsc.subcore_barrier()

        def sb_pair(p, carry):
            run_superblock(2 * p, 0)
            run_superblock(2 * p + 1, 1)
            return carry

        lax.fori_loop(0, NSB // 2, sb_pair, 0)
        plsc.subcore_barrier()
        pltpu.sync_copy(acc_sh.at[pl.ds(sid * ROWS_PER_TILE, ROWS_PER_TILE)],
                        out_hbm.at[cid, pl.ds(sid * ROWS_PER_TILE, ROWS_PER_TILE)])

        @pl.when(sid == 0)
        def _():
            pltpu.sync_copy(acc_sh.at[pl.ds(NS * ROWS_PER_TILE, TAIL_ROWS)],
                            out_hbm.at[cid, pl.ds(NS * ROWS_PER_TILE, TAIL_ROWS)])

    return spmm


_spmm_full = _make_spmm(EMB)
_spmm_half = _make_spmm(EMB, scale_groups=OUT // 16)


def kernel(user_feature, edge_index, edge_weight, gamma,
           W_num, b_num, W_cat, b_cat, W_des, b_des, W_tweet, b_tweet,
           W1, W2):
    f32 = jnp.float32
    wbd = jnp.zeros((D_IN, EMB), dtype=f32)
    wbd = wbd.at[0:5, 0:Q].set(W_num.T)
    wbd = wbd.at[5:16, Q:2 * Q].set(W_cat.T)
    wbd = wbd.at[16:784, 2 * Q:3 * Q].set(W_des.T)
    wbd = wbd.at[784:1552, 3 * Q:].set(W_tweet.T)
    bias = jnp.concatenate([b_num, b_cat, b_des, b_tweet]).reshape(1, EMB)

    pad = E_PAD - E
    pad_rows = (jnp.arange(pad, dtype=jnp.int32)) % N
    src = jnp.concatenate([edge_index[1].astype(jnp.int32), pad_rows])
    dst = jnp.concatenate(
        [edge_index[0].astype(jnp.int32), pad_rows]).reshape(
        NC * NS, K_PER_WORKER, CHUNK)
    ew = jnp.pad(edge_weight.astype(f32), (0, pad))
    zeros128 = jnp.zeros((N, EMB), dtype=f32)
    w2p = jnp.zeros((EMB, EMB), dtype=f32).at[:, :OUT].set(W2)

    tm = 1000
    h = pl.pallas_call(
        _enc_body,
        grid=(N // tm,),
        in_specs=[
            pl.BlockSpec((tm, D_IN), lambda i: (i, 0)),
            pl.BlockSpec((D_IN, EMB), lambda i: (0, 0)),
            pl.BlockSpec((1, EMB), lambda i: (0, 0)),
            pl.BlockSpec((EMB, EMB), lambda i: (0, 0)),
        ],
        out_specs=pl.BlockSpec((tm, EMB), lambda i: (i, 0)),
        out_shape=jax.ShapeDtypeStruct((N, EMB), f32),
    )(user_feature, wbd, bias, W1)

    p = _spmm_full(h, src, dst, ew, zeros128)

    m = pl.pallas_call(
        _mid_body,
        grid=(N // tm,),
        in_specs=[
            pl.BlockSpec((NC, tm, EMB), lambda i: (0, i, 0)),
            pl.BlockSpec((EMB, EMB), lambda i: (0, 0)),
        ],
        out_specs=pl.BlockSpec((tm, EMB), lambda i: (i, 0)),
        out_shape=jax.ShapeDtypeStruct((N, EMB), f32),
    )(p, w2p)

    q = _spmm_half(m, src, dst, ew, zeros128)

    z_mean = pl.pallas_call(
        _zmean_body,
        grid=(N // tm,),
        in_specs=[pl.BlockSpec((NC, tm, EMB), lambda i: (0, i, 0))],
        out_specs=pl.BlockSpec((tm, OUT), lambda i: (i, 0)),
        out_shape=jax.ShapeDtypeStruct((N, OUT), f32),
    )(q)

    gamma2d = jnp.asarray(gamma, dtype=f32).reshape(1, 1)
    tdec = 200
    reconstructions, clusters_distance = pl.pallas_call(
        _decoder_body,
        grid=(N // tdec,),
        in_specs=[
            pl.BlockSpec(memory_space=pltpu.SMEM),
            pl.BlockSpec((tdec, OUT), lambda i: (i, 0)),
            pl.BlockSpec((N, OUT), lambda i: (0, 0)),
        ],
        out_specs=[
            pl.BlockSpec((tdec, N), lambda i: (i, 0)),
            pl.BlockSpec((tdec, N), lambda i: (i, 0)),
        ],
        out_shape=[
            jax.ShapeDtypeStruct((N, N), f32),
            jax.ShapeDtypeStruct((N, N), f32),
        ],
    )(gamma2d, z_mean, z_mean)

    return reconstructions, clusters_distance, z_mean
